# baseline (speedup 1.0000x reference)
import functools

import jax
import jax.numpy as jnp
from jax import lax
from jax.experimental import pallas as pl
from jax.experimental.pallas import tpu as pltpu
from jax.experimental.pallas import tpu_sc as plsc

_R = 128
_N = 32768
_L = 16
_NC = 2
_NS = 16
_NW = _NC * _NS
_RPW = _R // _NW
_PAD = 16
_A = 4
_U = 32


def _insert(t0, t1, t2, v):
    hi0 = jnp.maximum(t0, v)
    lo0 = jnp.minimum(t0, v)
    hi1 = jnp.maximum(t1, lo0)
    lo1 = jnp.minimum(t1, lo0)
    hi2 = jnp.maximum(t2, lo1)
    return hi0, hi1, hi2


def _insert_low(t1, t2, v):
    hi1 = jnp.maximum(t1, v)
    lo1 = jnp.minimum(t1, v)
    hi2 = jnp.maximum(t2, lo1)
    return hi1, hi2


def _merge4(t0, t1, t2, v1, v2, v3, v4):
    m1 = jnp.maximum(v1, v2)
    n1 = jnp.minimum(v1, v2)
    m2 = jnp.maximum(v3, v4)
    n2 = jnp.minimum(v3, v4)
    a = jnp.maximum(m1, m2)
    b = jnp.minimum(m1, m2)
    c = jnp.maximum(n1, n2)
    t0, t1, t2 = _insert(t0, t1, t2, a)
    t1, t2 = _insert_low(t1, t2, b)
    t1, t2 = _insert_low(t1, t2, c)
    return t0, t1, t2


def _sc_topk3(a, b):
    mesh = plsc.VectorSubcoreMesh(core_axis_name="c", subcore_axis_name="s")

    @functools.partial(
        pl.kernel,
        mesh=mesh,
        compiler_params=pltpu.CompilerParams(needs_layout_passes=False),
        out_type=jax.ShapeDtypeStruct((_R, 3), jnp.float32),
        scratch_types=[
            pltpu.VMEM((_N // 2,), jnp.float32),
            pltpu.VMEM((_N // 2,), jnp.float32),
            pltpu.VMEM((_L,), jnp.float32),
            pltpu.VMEM((_L,), jnp.float32),
            pltpu.VMEM((_RPW, 3), jnp.float32),
            pltpu.SemaphoreType.DMA,
            pltpu.SemaphoreType.DMA,
            pltpu.SemaphoreType.DMA,
        ],
    )
    def k(a_hbm, b_hbm, out_hbm, buf0, buf1, bv, gs, ov, sem0, sem1, bsem):
        wid = lax.axis_index("s") * _NC + lax.axis_index("c")
        row0 = wid * _RPW
        bcopy = pltpu.async_copy(b_hbm, bv.at[pl.ds(0, 3)], bsem)
        idx = lax.iota(jnp.int32, _L)
        neg = jnp.full((_L,), -jnp.inf, jnp.float32)
        bufs = (buf0, buf1)
        sems = (sem0, sem1)
        copies = [None, None]
        _H = _N // 2
        copies[0] = pltpu.async_copy(
            a_hbm.at[row0, pl.ds(0, _H)], buf0, sem0)
        bvec = None
        init = (neg,) * (3 * _A)
        ts = init
        for h in range(2 * _RPW):
            if h + 1 < 2 * _RPW:
                nb = (h + 1) % 2
                copies[nb] = pltpu.async_copy(
                    a_hbm.at[row0 + (h + 1) // 2,
                             pl.ds(((h + 1) % 2) * _H, _H)],
                    bufs[nb], sems[nb])
            cb = h % 2
            copies[cb].wait()
            buf = bufs[cb]

            def step(i, c):
                ts_ = list(c)
                o = i * (_U * _L)
                for g in range(_U // 4):
                    vs = [buf[pl.ds(o + (4 * g + j) * _L, _L)]
                          for j in range(4)]
                    a_ = g % _A
                    t0, t1, t2 = _merge4(ts_[3 * a_], ts_[3 * a_ + 1],
                                         ts_[3 * a_ + 2], *vs)
                    ts_[3 * a_] = t0
                    ts_[3 * a_ + 1] = t1
                    ts_[3 * a_ + 2] = t2
                return tuple(ts_)

            ts = lax.fori_loop(0, _H // (_U * _L), step, ts)
            if h % 2 == 0:
                continue
            row = h // 2
            if bvec is None:
                bcopy.wait()
                bvec = bv[...]
            t0, t1, t2 = ts[0], ts[1], ts[2]
            for a_ in range(1, _A):
                t0, t1, t2 = _insert(t0, t1, t2, ts[3 * a_])
                t1, t2 = _insert_low(t1, t2, ts[3 * a_ + 1])
                t1, t2 = _insert_low(t1, t2, ts[3 * a_ + 2])
            s0 = lax.sort(t0)
            s1 = lax.sort(t1)
            s2 = lax.sort(t2)
            gs[...] = s1
            g1 = plsc.load_gather(gs, [(idx + 3) & 15])
            gs[...] = s2
            g2 = plsc.load_gather(gs, [(idx + 6) & 15])
            cand = jnp.where(idx >= 13, s0,
                             jnp.where(idx >= 10, g1,
                                       jnp.where(idx >= 7, g2, neg)))
            res = lax.rev(lax.sort(cand), (0,))
            rowv = jnp.full((_L,), row, jnp.int32)
            plsc.store_scatter(ov, [rowv, idx], res * 2.0 + bvec,
                               mask=idx < 3)
            ts = init
        pltpu.sync_copy(ov, out_hbm.at[pl.ds(row0, _RPW)])

    return k(a, b)


def kernel(a, b):
    return _sc_topk3(a, b)


# speedup vs baseline: 1.0404x; 1.0404x over previous
import functools

import jax
import jax.numpy as jnp
from jax import lax
from jax.experimental import pallas as pl
from jax.experimental.pallas import tpu as pltpu
from jax.experimental.pallas import tpu_sc as plsc

_R = 128
_N = 32768
_L = 16
_NC = 2
_NS = 16
_NW = _NC * _NS
_RPW = _R // _NW
_PAD = 16
_A = 4
_U = 16


def _insert(t0, t1, t2, v):
    hi0 = jnp.maximum(t0, v)
    lo0 = jnp.minimum(t0, v)
    hi1 = jnp.maximum(t1, lo0)
    lo1 = jnp.minimum(t1, lo0)
    hi2 = jnp.maximum(t2, lo1)
    return hi0, hi1, hi2


def _insert_low(t1, t2, v):
    hi1 = jnp.maximum(t1, v)
    lo1 = jnp.minimum(t1, v)
    hi2 = jnp.maximum(t2, lo1)
    return hi1, hi2


def _merge4(t0, t1, t2, v1, v2, v3, v4):
    m1 = jnp.maximum(v1, v2)
    n1 = jnp.minimum(v1, v2)
    m2 = jnp.maximum(v3, v4)
    n2 = jnp.minimum(v3, v4)
    a = jnp.maximum(m1, m2)
    b = jnp.minimum(m1, m2)
    c = jnp.maximum(n1, n2)
    t0, t1, t2 = _insert(t0, t1, t2, a)
    t1, t2 = _insert_low(t1, t2, b)
    t1, t2 = _insert_low(t1, t2, c)
    return t0, t1, t2


def _sc_topk3(a, b):
    mesh = plsc.VectorSubcoreMesh(core_axis_name="c", subcore_axis_name="s")

    @functools.partial(
        pl.kernel,
        mesh=mesh,
        compiler_params=pltpu.CompilerParams(needs_layout_passes=False),
        out_type=jax.ShapeDtypeStruct((_R, 3), jnp.float32),
        scratch_types=[
            pltpu.VMEM((_N // 2,), jnp.float32),
            pltpu.VMEM((_N // 2,), jnp.float32),
            pltpu.VMEM((_L,), jnp.float32),
            pltpu.VMEM((_L,), jnp.float32),
            pltpu.VMEM((_RPW, 3), jnp.float32),
            pltpu.SemaphoreType.DMA,
            pltpu.SemaphoreType.DMA,
            pltpu.SemaphoreType.DMA,
        ],
    )
    def k(a_hbm, b_hbm, out_hbm, buf0, buf1, bv, gs, ov, sem0, sem1, bsem):
        wid = lax.axis_index("s") * _NC + lax.axis_index("c")
        row0 = wid * _RPW
        bcopy = pltpu.async_copy(b_hbm, bv.at[pl.ds(0, 3)], bsem)
        idx = lax.iota(jnp.int32, _L)
        neg = jnp.full((_L,), -jnp.inf, jnp.float32)
        bufs = (buf0, buf1)
        sems = (sem0, sem1)
        copies = [None, None]
        _H = _N // 2
        copies[0] = pltpu.async_copy(
            a_hbm.at[row0, pl.ds(0, _H)], buf0, sem0)
        bvec = None
        init = (neg,) * (3 * _A)
        ts = init
        for h in range(2 * _RPW):
            if h + 1 < 2 * _RPW:
                nb = (h + 1) % 2
                copies[nb] = pltpu.async_copy(
                    a_hbm.at[row0 + (h + 1) // 2,
                             pl.ds(((h + 1) % 2) * _H, _H)],
                    bufs[nb], sems[nb])
            cb = h % 2
            copies[cb].wait()
            buf = bufs[cb]

            def step(i, c):
                ts_ = list(c)
                o = i * (_U * _L)
                for g in range(_U // 4):
                    vs = [buf[pl.ds(o + (4 * g + j) * _L, _L)]
                          for j in range(4)]
                    a_ = g % _A
                    t0, t1, t2 = _merge4(ts_[3 * a_], ts_[3 * a_ + 1],
                                         ts_[3 * a_ + 2], *vs)
                    ts_[3 * a_] = t0
                    ts_[3 * a_ + 1] = t1
                    ts_[3 * a_ + 2] = t2
                return tuple(ts_)

            ts = lax.fori_loop(0, _H // (_U * _L), step, ts)
            if h % 2 == 0:
                continue
            row = h // 2
            if bvec is None:
                bcopy.wait()
                bvec = bv[...]
            t0, t1, t2 = ts[0], ts[1], ts[2]
            for a_ in range(1, _A):
                t0, t1, t2 = _insert(t0, t1, t2, ts[3 * a_])
                t1, t2 = _insert_low(t1, t2, ts[3 * a_ + 1])
                t1, t2 = _insert_low(t1, t2, ts[3 * a_ + 2])
            s0 = lax.sort(t0)
            s1 = lax.sort(t1)
            s2 = lax.sort(t2)
            gs[...] = s1
            g1 = plsc.load_gather(gs, [(idx + 3) & 15])
            gs[...] = s2
            g2 = plsc.load_gather(gs, [(idx + 6) & 15])
            cand = jnp.where(idx >= 13, s0,
                             jnp.where(idx >= 10, g1,
                                       jnp.where(idx >= 7, g2, neg)))
            res = lax.rev(lax.sort(cand), (0,))
            rowv = jnp.full((_L,), row, jnp.int32)
            plsc.store_scatter(ov, [rowv, idx], res * 2.0 + bvec,
                               mask=idx < 3)
            ts = init
        pltpu.sync_copy(ov, out_hbm.at[pl.ds(row0, _RPW)])

    return k(a, b)


def kernel(a, b):
    return _sc_topk3(a, b)


# speedup vs baseline: 1.0421x; 1.0016x over previous
import functools

import jax
import jax.numpy as jnp
from jax import lax
from jax.experimental import pallas as pl
from jax.experimental.pallas import tpu as pltpu
from jax.experimental.pallas import tpu_sc as plsc

_R = 128
_N = 32768
_L = 16
_NC = 2
_NS = 16
_NW = _NC * _NS
_RPW = _R // _NW
_A = 4
_U = 16


def _insert(t0, t1, t2, v):
    hi0 = jnp.maximum(t0, v)
    lo0 = jnp.minimum(t0, v)
    hi1 = jnp.maximum(t1, lo0)
    lo1 = jnp.minimum(t1, lo0)
    hi2 = jnp.maximum(t2, lo1)
    return hi0, hi1, hi2


def _insert_low(t1, t2, v):
    hi1 = jnp.maximum(t1, v)
    lo1 = jnp.minimum(t1, v)
    hi2 = jnp.maximum(t2, lo1)
    return hi1, hi2


def _merge4(t0, t1, t2, v1, v2, v3, v4):
    m1 = jnp.maximum(v1, v2)
    n1 = jnp.minimum(v1, v2)
    m2 = jnp.maximum(v3, v4)
    n2 = jnp.minimum(v3, v4)
    a = jnp.maximum(m1, m2)
    b = jnp.minimum(m1, m2)
    c = jnp.maximum(n1, n2)
    t0, t1, t2 = _insert(t0, t1, t2, a)
    t1, t2 = _insert_low(t1, t2, b)
    t1, t2 = _insert_low(t1, t2, c)
    return t0, t1, t2


def _sc_topk3(a, b):
    mesh = plsc.VectorSubcoreMesh(core_axis_name="c", subcore_axis_name="s")

    @functools.partial(
        pl.kernel,
        mesh=mesh,
        compiler_params=pltpu.CompilerParams(needs_layout_passes=False),
        out_type=jax.ShapeDtypeStruct((_R, 3), jnp.float32),
        scratch_types=[
            pltpu.VMEM((_N // 2,), jnp.float32),
            pltpu.VMEM((_N // 2,), jnp.float32),
            pltpu.VMEM((_L,), jnp.float32),
            pltpu.VMEM((_L,), jnp.float32),
            pltpu.VMEM((_RPW, 3), jnp.float32),
            pltpu.SemaphoreType.DMA,
            pltpu.SemaphoreType.DMA,
            pltpu.SemaphoreType.DMA,
        ],
    )
    def k(a_hbm, b_hbm, out_hbm, buf0, buf1, bv, gs, ov, sem0, sem1, bsem):
        wid = lax.axis_index("s") * _NC + lax.axis_index("c")
        row0 = wid * _RPW
        bcopy = pltpu.async_copy(b_hbm, bv.at[pl.ds(0, 3)], bsem)
        idx = lax.iota(jnp.int32, _L)
        neg = jnp.full((_L,), -jnp.inf, jnp.float32)
        bufs = (buf0, buf1)
        sems = (sem0, sem1)
        copies = [None, None]
        _H = _N // 2
        copies[0] = pltpu.async_copy(
            a_hbm.at[row0, pl.ds(0, _H)], buf0, sem0)
        bvec = None
        init = (neg,) * (3 * _A)
        ts = init
        for h in range(2 * _RPW):
            if h + 1 < 2 * _RPW:
                nb = (h + 1) % 2
                copies[nb] = pltpu.async_copy(
                    a_hbm.at[row0 + (h + 1) // 2,
                             pl.ds(((h + 1) % 2) * _H, _H)],
                    bufs[nb], sems[nb])
            cb = h % 2
            copies[cb].wait()
            buf = bufs[cb]

            def step(i, c):
                ts_ = list(c)
                o = i * (_U * _L)
                for g in range(_U // 4):
                    vs = [buf[pl.ds(o + (4 * g + j) * _L, _L)]
                          for j in range(4)]
                    a_ = g % _A
                    t0, t1, t2 = _merge4(ts_[3 * a_], ts_[3 * a_ + 1],
                                         ts_[3 * a_ + 2], *vs)
                    ts_[3 * a_] = t0
                    ts_[3 * a_ + 1] = t1
                    ts_[3 * a_ + 2] = t2
                return tuple(ts_)

            ts = lax.fori_loop(0, _H // (_U * _L), step, ts)
            if h % 2 == 0:
                continue
            row = h // 2
            if bvec is None:
                bcopy.wait()
                bvec = bv[...]
            t0, t1, t2 = ts[0], ts[1], ts[2]
            for a_ in range(1, _A):
                t0, t1, t2 = _insert(t0, t1, t2, ts[3 * a_])
                t1, t2 = _insert_low(t1, t2, ts[3 * a_ + 1])
                t1, t2 = _insert_low(t1, t2, ts[3 * a_ + 2])
            s0 = lax.sort(t0)
            s1 = lax.sort(t1)
            s2 = lax.sort(t2)
            gs[...] = s1
            g1 = plsc.load_gather(gs, [(idx + 3) & 15])
            gs[...] = s2
            g2 = plsc.load_gather(gs, [(idx + 6) & 15])
            cand = jnp.where(idx >= 13, s0,
                             jnp.where(idx >= 10, g1,
                                       jnp.where(idx >= 7, g2, neg)))
            res = lax.rev(lax.sort(cand), (0,))
            rowv = jnp.full((_L,), row, jnp.int32)
            plsc.store_scatter(ov, [rowv, idx], res * 2.0 + bvec,
                               mask=idx < 3)
            ts = init
        pltpu.sync_copy(ov, out_hbm.at[pl.ds(row0, _RPW)])

    return k(a, b)


def kernel(a, b):
    return _sc_topk3(a, b)
